# baseline (speedup 1.0000x reference)
import jax
import jax.numpy as jnp
from jax.experimental import pallas as pl
from jax.experimental.pallas import tpu as pltpu

_N_CHUNKS = 8
_N_BUFS = 4


def _body(srow_ref, scol_ref, ct_hbm, ind_ref, t_ref, out_hbm, indout_ref,
          *rest):
    bufs = rest[:_N_BUFS]
    lsems = rest[_N_BUFS:_N_BUFS + _N_CHUNKS]
    ssems = rest[_N_BUFS + _N_CHUNKS:]
    row = srow_ref[0]
    col = scol_ref[0]
    n_det, n_cols = ct_hbm.shape
    rchunk = n_det // _N_CHUNKS

    def load(i):
        return pltpu.make_async_copy(
            ct_hbm.at[pl.ds(i * rchunk, rchunk), :], bufs[i % _N_BUFS], lsems[i]
        )

    def store(i):
        return pltpu.make_async_copy(
            bufs[i % _N_BUFS], out_hbm.at[pl.ds(i * rchunk, rchunk), :], ssems[i]
        )

    for i in range(_N_BUFS):
        load(i).start()

    lanes = jax.lax.broadcasted_iota(jnp.int32, ind_ref.shape, 1)
    indout_ref[...] = ind_ref[...] + (lanes == row).astype(jnp.int32)

    for i in range(_N_CHUNKS):
        load(i).wait()
        base = i * rchunk
        hit = jnp.logical_and(row >= base, row < base + rchunk)

        @pl.when(hit)
        def _():
            buf = bufs[i % _N_BUFS]
            rows = jax.lax.broadcasted_iota(jnp.int32, buf.shape, 0) + base
            cols = jax.lax.broadcasted_iota(jnp.int32, buf.shape, 1)
            mask = jnp.logical_and(rows == row, cols == col)
            buf[...] = jnp.where(mask, t_ref[0, 0], buf[...])

        store(i).start()
        if i + _N_BUFS < _N_CHUNKS:
            store(i).wait()
            load(i + _N_BUFS).start()

    for i in range(max(0, _N_CHUNKS - _N_BUFS), _N_CHUNKS):
        store(i).wait()


def kernel(clicktimes, indices, idx, t):
    n_det, n_cols = clicktimes.shape
    rchunk = n_det // _N_CHUNKS
    row = jnp.asarray(idx, jnp.int32).reshape(1)
    col = jnp.take(indices, jnp.asarray(idx, jnp.int32)).reshape(1)
    ind2d = indices.reshape(1, n_det)
    t2d = jnp.asarray(t, jnp.float32).reshape(1, 1)

    out, indout = pl.pallas_call(
        _body,
        grid_spec=pltpu.PrefetchScalarGridSpec(
            num_scalar_prefetch=2,
            grid=(),
            in_specs=[
                pl.BlockSpec(memory_space=pltpu.HBM),
                pl.BlockSpec(memory_space=pltpu.VMEM),
                pl.BlockSpec(memory_space=pltpu.VMEM),
            ],
            out_specs=[
                pl.BlockSpec(memory_space=pltpu.HBM),
                pl.BlockSpec(memory_space=pltpu.VMEM),
            ],
            scratch_shapes=[pltpu.VMEM((rchunk, n_cols), jnp.float32)
                            for _ in range(_N_BUFS)]
            + [pltpu.SemaphoreType.DMA] * (2 * _N_CHUNKS),
        ),
        out_shape=[
            jax.ShapeDtypeStruct((n_det, n_cols), clicktimes.dtype),
            jax.ShapeDtypeStruct((1, n_det), indices.dtype),
        ],
    )(row, col, clicktimes, ind2d, t2d)
    return (out, indout.reshape(n_det))


# speedup vs baseline: 1.1401x; 1.1401x over previous
import jax
import jax.numpy as jnp
from jax.experimental import pallas as pl
from jax.experimental.pallas import tpu as pltpu

_N_CHUNKS = 8
_N_BUFS = 8


def _body(srow_ref, scol_ref, ct_hbm, ind_ref, t_ref, out_hbm, indout_ref,
          *rest):
    bufs = rest[:_N_BUFS]
    lsems = rest[_N_BUFS:_N_BUFS + _N_CHUNKS]
    ssems = rest[_N_BUFS + _N_CHUNKS:]
    row = srow_ref[0]
    col = scol_ref[0]
    n_det, n_cols = ct_hbm.shape
    rchunk = n_det // _N_CHUNKS

    def load(i):
        return pltpu.make_async_copy(
            ct_hbm.at[pl.ds(i * rchunk, rchunk), :], bufs[i % _N_BUFS], lsems[i]
        )

    def store(i):
        return pltpu.make_async_copy(
            bufs[i % _N_BUFS], out_hbm.at[pl.ds(i * rchunk, rchunk), :], ssems[i]
        )

    for i in range(_N_BUFS):
        load(i).start()

    lanes = jax.lax.broadcasted_iota(jnp.int32, ind_ref.shape, 1)
    indout_ref[...] = ind_ref[...] + (lanes == row).astype(jnp.int32)

    for i in range(_N_CHUNKS):
        load(i).wait()
        base = i * rchunk
        hit = jnp.logical_and(row >= base, row < base + rchunk)

        @pl.when(hit)
        def _():
            buf = bufs[i % _N_BUFS]
            rows = jax.lax.broadcasted_iota(jnp.int32, buf.shape, 0) + base
            cols = jax.lax.broadcasted_iota(jnp.int32, buf.shape, 1)
            mask = jnp.logical_and(rows == row, cols == col)
            buf[...] = jnp.where(mask, t_ref[0, 0], buf[...])

        store(i).start()

    for i in range(_N_CHUNKS):
        store(i).wait()


def kernel(clicktimes, indices, idx, t):
    n_det, n_cols = clicktimes.shape
    rchunk = n_det // _N_CHUNKS
    row = jnp.asarray(idx, jnp.int32).reshape(1)
    col = jnp.take(indices, jnp.asarray(idx, jnp.int32)).reshape(1)
    ind2d = indices.reshape(1, n_det)
    t2d = jnp.asarray(t, jnp.float32).reshape(1, 1)

    out, indout = pl.pallas_call(
        _body,
        grid_spec=pltpu.PrefetchScalarGridSpec(
            num_scalar_prefetch=2,
            grid=(),
            in_specs=[
                pl.BlockSpec(memory_space=pltpu.HBM),
                pl.BlockSpec(memory_space=pltpu.VMEM),
                pl.BlockSpec(memory_space=pltpu.VMEM),
            ],
            out_specs=[
                pl.BlockSpec(memory_space=pltpu.HBM),
                pl.BlockSpec(memory_space=pltpu.VMEM),
            ],
            scratch_shapes=[pltpu.VMEM((rchunk, n_cols), jnp.float32)
                            for _ in range(_N_BUFS)]
            + [pltpu.SemaphoreType.DMA] * (2 * _N_CHUNKS),
        ),
        out_shape=[
            jax.ShapeDtypeStruct((n_det, n_cols), clicktimes.dtype),
            jax.ShapeDtypeStruct((1, n_det), indices.dtype),
        ],
    )(row, col, clicktimes, ind2d, t2d)
    return (out, indout.reshape(n_det))


# speedup vs baseline: 1.1514x; 1.0099x over previous
import jax
import jax.numpy as jnp
from jax.experimental import pallas as pl
from jax.experimental.pallas import tpu as pltpu

_N_CHUNKS = 16
_N_BUFS = 16


def _body(srow_ref, scol_ref, ct_hbm, ind_ref, t_ref, out_hbm, indout_ref,
          *rest):
    bufs = rest[:_N_BUFS]
    lsems = rest[_N_BUFS:_N_BUFS + _N_CHUNKS]
    ssems = rest[_N_BUFS + _N_CHUNKS:]
    row = srow_ref[0]
    col = scol_ref[0]
    n_det, n_cols = ct_hbm.shape
    rchunk = n_det // _N_CHUNKS

    def load(i):
        return pltpu.make_async_copy(
            ct_hbm.at[pl.ds(i * rchunk, rchunk), :], bufs[i % _N_BUFS], lsems[i]
        )

    def store(i):
        return pltpu.make_async_copy(
            bufs[i % _N_BUFS], out_hbm.at[pl.ds(i * rchunk, rchunk), :], ssems[i]
        )

    for i in range(_N_BUFS):
        load(i).start()

    lanes = jax.lax.broadcasted_iota(jnp.int32, ind_ref.shape, 1)
    indout_ref[...] = ind_ref[...] + (lanes == row).astype(jnp.int32)

    for i in range(_N_CHUNKS):
        load(i).wait()
        base = i * rchunk
        hit = jnp.logical_and(row >= base, row < base + rchunk)

        @pl.when(hit)
        def _():
            buf = bufs[i % _N_BUFS]
            rows = jax.lax.broadcasted_iota(jnp.int32, buf.shape, 0) + base
            cols = jax.lax.broadcasted_iota(jnp.int32, buf.shape, 1)
            mask = jnp.logical_and(rows == row, cols == col)
            buf[...] = jnp.where(mask, t_ref[0, 0], buf[...])

        store(i).start()

    for i in range(_N_CHUNKS):
        store(i).wait()


def kernel(clicktimes, indices, idx, t):
    n_det, n_cols = clicktimes.shape
    rchunk = n_det // _N_CHUNKS
    row = jnp.asarray(idx, jnp.int32).reshape(1)
    col = jnp.take(indices, jnp.asarray(idx, jnp.int32)).reshape(1)
    ind2d = indices.reshape(1, n_det)
    t2d = jnp.asarray(t, jnp.float32).reshape(1, 1)

    out, indout = pl.pallas_call(
        _body,
        grid_spec=pltpu.PrefetchScalarGridSpec(
            num_scalar_prefetch=2,
            grid=(),
            in_specs=[
                pl.BlockSpec(memory_space=pltpu.HBM),
                pl.BlockSpec(memory_space=pltpu.VMEM),
                pl.BlockSpec(memory_space=pltpu.VMEM),
            ],
            out_specs=[
                pl.BlockSpec(memory_space=pltpu.HBM),
                pl.BlockSpec(memory_space=pltpu.VMEM),
            ],
            scratch_shapes=[pltpu.VMEM((rchunk, n_cols), jnp.float32)
                            for _ in range(_N_BUFS)]
            + [pltpu.SemaphoreType.DMA] * (2 * _N_CHUNKS),
        ),
        out_shape=[
            jax.ShapeDtypeStruct((n_det, n_cols), clicktimes.dtype),
            jax.ShapeDtypeStruct((1, n_det), indices.dtype),
        ],
    )(row, col, clicktimes, ind2d, t2d)
    return (out, indout.reshape(n_det))
